# Initial kernel scaffold; baseline (speedup 1.0000x reference)
import jax
import jax.numpy as jnp
from jax.experimental import pallas as pl


def kernel(x, edge_index, W1l, W1r, b1, W2l, W2r, b2, W3l, W3r, b3, W4l, W4r, b4):
    raise NotImplementedError("write your pallas kernel here")


# speedup vs baseline: 4.6513x; 4.6513x over previous
import functools

import jax
import jax.numpy as jnp
from jax import lax
from jax.experimental import pallas as pl
from jax.experimental.pallas import tpu as pltpu
from jax.experimental.pallas import tpu_sc as plsc

N_NODES = 10000
D_IN = 128
D_OUT = 40
D_OUT_PAD = 128
N_EDGES = 320000
NC = 2
NS = 16
CH = 80
E_PER_TILE = N_EDGES // (NC * NS)
N_CHUNKS = E_PER_TILE // CH
N_PAD = 10240
ROWS_PER_SUB = N_PAD // NS
CNT_W = 128


def _make_agg(d):
  mesh = plsc.VectorSubcoreMesh(core_axis_name="c", subcore_axis_name="s")

  @functools.partial(
      pl.kernel, mesh=mesh,
      out_type=jax.ShapeDtypeStruct((NC, N_PAD, d), jnp.float32),
      scratch_types=[
          pltpu.VMEM_SHARED((N_PAD, d), jnp.float32),
          pltpu.VMEM((CH,), jnp.int32),
          pltpu.VMEM((CH,), jnp.int32),
          pltpu.VMEM((CH, d), jnp.float32),
          pltpu.SemaphoreType.DMA,
      ])
  def agg(table, src, dst, out, acc, srcv, dstv, rows, sem):
    c = lax.axis_index("c")
    s = lax.axis_index("s")
    z16 = jnp.zeros((16,), jnp.float32)

    def zrow(i, carry):
      for j in range(d // 16):
        rows[i, pl.ds(j * 16, 16)] = z16
      return carry
    lax.fori_loop(0, CH, zrow, 0)

    base_r = pl.multiple_of(s * ROWS_PER_SUB, 8)
    nfull = ROWS_PER_SUB // CH
    for k in range(nfull):
      pltpu.sync_copy(rows, acc.at[pl.ds(base_r + k * CH, CH)])
    plsc.subcore_barrier()

    ebase = (c * NS + s) * E_PER_TILE

    def chunk(k, carry):
      b = pl.multiple_of(ebase + k * CH, 8)
      pltpu.sync_copy(src.at[pl.ds(b, CH)], srcv)
      pltpu.sync_copy(dst.at[pl.ds(b, CH)], dstv)
      pltpu.async_copy(table.at[srcv], rows, sem).wait()
      pltpu.sync_copy(rows, acc.at[dstv], add=True)
      return carry
    lax.fori_loop(0, N_CHUNKS, chunk, 0)
    plsc.subcore_barrier()

    for k in range(nfull):
      pltpu.sync_copy(acc.at[pl.ds(base_r + k * CH, CH)], rows)
      pltpu.sync_copy(rows, out.at[c, pl.ds(base_r + k * CH, CH)])

  return agg


def _make_counts():
  mesh = plsc.VectorSubcoreMesh(core_axis_name="c", subcore_axis_name="s")

  @functools.partial(
      pl.kernel, mesh=mesh,
      out_type=jax.ShapeDtypeStruct((NC, N_PAD, CNT_W), jnp.float32),
      scratch_types=[
          pltpu.VMEM_SHARED((N_PAD, CNT_W), jnp.float32),
          pltpu.VMEM((CH,), jnp.int32),
          pltpu.VMEM((CH, CNT_W), jnp.float32),
          pltpu.VMEM((CH, CNT_W), jnp.float32),
      ])
  def cnts(dst, out, acc, dstv, ones, zbuf):
    c = lax.axis_index("c")
    s = lax.axis_index("s")
    z16 = jnp.zeros((16,), jnp.float32)
    o16 = jnp.ones((16,), jnp.float32)

    def frow(i, carry):
      for j in range(CNT_W // 16):
        zbuf[i, pl.ds(j * 16, 16)] = z16
        ones[i, pl.ds(j * 16, 16)] = o16
      return carry
    lax.fori_loop(0, CH, frow, 0)

    base_r = pl.multiple_of(s * ROWS_PER_SUB, 8)
    nfull = ROWS_PER_SUB // CH
    for k in range(nfull):
      pltpu.sync_copy(zbuf, acc.at[pl.ds(base_r + k * CH, CH)])
    plsc.subcore_barrier()

    ebase = (c * NS + s) * E_PER_TILE

    def chunk(k, carry):
      b = pl.multiple_of(ebase + k * CH, 8)
      pltpu.sync_copy(dst.at[pl.ds(b, CH)], dstv)
      pltpu.sync_copy(ones, acc.at[dstv], add=True)
      return carry
    lax.fori_loop(0, N_CHUNKS, chunk, 0)
    plsc.subcore_barrier()

    for k in range(nfull):
      pltpu.sync_copy(acc.at[pl.ds(base_r + k * CH, CH)], zbuf)
      pltpu.sync_copy(zbuf, out.at[c, pl.ds(base_r + k * CH, CH)])

  return cnts


_ROWS_BLK = 1000


def _tc_in_body(x, wl, wr, b, hl, hr):
  xb = x[...]
  hl[...] = jnp.dot(xb, wl[...], preferred_element_type=jnp.float32)
  hr[...] = jnp.dot(xb, wr[...], preferred_element_type=jnp.float32) + b[...]


def _tc_in(x, Wl, Wr, b):
  di = x.shape[1]
  do = Wl.shape[1]
  grid = (N_NODES // _ROWS_BLK,)
  return pl.pallas_call(
      _tc_in_body,
      grid=grid,
      in_specs=[
          pl.BlockSpec((_ROWS_BLK, di), lambda i: (i, 0)),
          pl.BlockSpec((di, do), lambda i: (0, 0)),
          pl.BlockSpec((di, do), lambda i: (0, 0)),
          pl.BlockSpec((1, do), lambda i: (0, 0)),
      ],
      out_specs=[
          pl.BlockSpec((_ROWS_BLK, do), lambda i: (i, 0)),
          pl.BlockSpec((_ROWS_BLK, do), lambda i: (i, 0)),
      ],
      out_shape=[jax.ShapeDtypeStruct((N_NODES, do), jnp.float32)] * 2,
  )(x, Wl, Wr, b.reshape(1, -1))


def _tc_mid_body(p0, p1, c0, c1, hrp, wl, wr, b, hl, hr):
  cnt = jnp.maximum(c0[...][:, :1] + c1[...][:, :1], 1.0)
  z = (p0[...] + p1[...]) / cnt + hrp[...]
  h = jnp.where(z > 0, z, jnp.exp(jnp.minimum(z, 0.0)) - 1.0)
  hl[...] = jnp.dot(h, wl[...], preferred_element_type=jnp.float32)
  hr[...] = jnp.dot(h, wr[...], preferred_element_type=jnp.float32) + b[...]


def _tc_mid(p0, p1, c0, c1, hrp, Wl, Wr, b):
  di = p0.shape[1]
  do = Wl.shape[1]
  grid = (N_NODES // _ROWS_BLK,)
  return pl.pallas_call(
      _tc_mid_body,
      grid=grid,
      in_specs=[
          pl.BlockSpec((_ROWS_BLK, di), lambda i: (i, 0)),
          pl.BlockSpec((_ROWS_BLK, di), lambda i: (i, 0)),
          pl.BlockSpec((_ROWS_BLK, CNT_W), lambda i: (i, 0)),
          pl.BlockSpec((_ROWS_BLK, CNT_W), lambda i: (i, 0)),
          pl.BlockSpec((_ROWS_BLK, di), lambda i: (i, 0)),
          pl.BlockSpec((di, do), lambda i: (0, 0)),
          pl.BlockSpec((di, do), lambda i: (0, 0)),
          pl.BlockSpec((1, do), lambda i: (0, 0)),
      ],
      out_specs=[
          pl.BlockSpec((_ROWS_BLK, do), lambda i: (i, 0)),
          pl.BlockSpec((_ROWS_BLK, do), lambda i: (i, 0)),
      ],
      out_shape=[jax.ShapeDtypeStruct((N_NODES, do), jnp.float32)] * 2,
  )(p0, p1, c0, c1, hrp, Wl, Wr, b.reshape(1, -1))


def _tc_out_body(p0, p1, c0, c1, hrp, o):
  cnt = jnp.maximum(c0[...][:, :1] + c1[...][:, :1], 1.0)
  z = (p0[...][:, :D_OUT] + p1[...][:, :D_OUT]) / cnt + hrp[...][:, :D_OUT]
  m = jnp.max(z, axis=-1, keepdims=True)
  zs = z - m
  o[...] = zs - jnp.log(jnp.sum(jnp.exp(zs), axis=-1, keepdims=True))


def _tc_out(p0, p1, c0, c1, hrp):
  grid = (N_NODES // _ROWS_BLK,)
  return pl.pallas_call(
      _tc_out_body,
      grid=grid,
      in_specs=[
          pl.BlockSpec((_ROWS_BLK, D_OUT_PAD), lambda i: (i, 0)),
          pl.BlockSpec((_ROWS_BLK, D_OUT_PAD), lambda i: (i, 0)),
          pl.BlockSpec((_ROWS_BLK, CNT_W), lambda i: (i, 0)),
          pl.BlockSpec((_ROWS_BLK, CNT_W), lambda i: (i, 0)),
          pl.BlockSpec((_ROWS_BLK, D_OUT_PAD), lambda i: (i, 0)),
      ],
      out_specs=pl.BlockSpec((_ROWS_BLK, D_OUT), lambda i: (i, 0)),
      out_shape=jax.ShapeDtypeStruct((N_NODES, D_OUT), jnp.float32),
  )(p0, p1, c0, c1, hrp)


def kernel(x, edge_index, W1l, W1r, b1, W2l, W2r, b2, W3l, W3r, b3,
           W4l, W4r, b4):
  src = edge_index[0].astype(jnp.int32)
  dst = edge_index[1].astype(jnp.int32)

  agg128 = _make_agg(D_IN)
  counts = _make_counts()

  cp = counts(dst)
  c0, c1 = cp[0], cp[1]

  W4l_p = jnp.pad(W4l, ((0, 0), (0, D_OUT_PAD - D_OUT)))
  W4r_p = jnp.pad(W4r, ((0, 0), (0, D_OUT_PAD - D_OUT)))
  b4_p = jnp.pad(b4, (0, D_OUT_PAD - D_OUT))

  hl, hr = _tc_in(x, W1l, W1r, b1)
  p = agg128(hl, src, dst)
  hl, hr = _tc_mid(p[0], p[1], c0, c1, hr, W2l, W2r, b2)
  p = agg128(hl, src, dst)
  hl, hr = _tc_mid(p[0], p[1], c0, c1, hr, W3l, W3r, b3)
  p = agg128(hl, src, dst)
  hl, hr = _tc_mid(p[0], p[1], c0, c1, hr, W4l_p, W4r_p, b4_p)
  p = agg128(hl, src, dst)
  return _tc_out(p[0], p[1], c0, c1, hr)
